# baseline (speedup 1.0000x reference)
import functools
import math

import jax
import jax.numpy as jnp
from jax import lax
from jax.experimental import pallas as pl
from jax.experimental.pallas import tpu as pltpu
from jax.experimental.pallas import tpu_sc as plsc

_N = 2097152
_C, _H, _W = 16, 256, 256
_HW = _H * _W

_PI = math.pi
_HALF_PI = math.pi / 2.0

_ATAN7 = (
    0.9999997406246051, -0.33331371051979747, 0.1996335468228588,
    -0.13991705989210754, 0.09853834258325642, -0.05880045262807415,
    0.02386837377624885, -0.004610804113185697,
)
_ASIN5 = (
    0.999999990618061, 0.1666682132063969, 0.07493977116698855,
    0.04556902120968854, 0.023874653067893874, 0.04248026658741398,
)


def _poly(coefs, s):
    acc = jnp.full_like(s, coefs[-1])
    for c in coefs[-2::-1]:
        acc = acc * s + c
    return acc


def _tc_coords_body(d_ref, i00_ref, i10_ref, i01_ref, i11_ref,
                    w00_ref, w10_ref, w01_ref, w11_ref):
    x = d_ref[0]
    y = d_ref[1]
    z = d_ref[2]

    ax = jnp.abs(x)
    az = jnp.abs(z)
    num = jnp.minimum(ax, az)
    den = jnp.maximum(ax, az)
    den = jnp.where(den == 0.0, 1.0, den)
    t = num / den
    p = t * _poly(_ATAN7, t * t)
    r = jnp.where(ax > az, _HALF_PI - p, p)
    r = jnp.where(z < 0.0, _PI - r, r)
    theta = jnp.where(x < 0.0, -r, r)

    ycl = jnp.clip(y, -1.0, 1.0)
    ay = jnp.abs(ycl)
    big = ay > 0.5
    s2 = jnp.where(big, (1.0 - ay) * 0.5, ay * ay)
    h = _poly(_ASIN5, s2)
    q = jnp.sqrt(s2)
    r2 = jnp.where(big, _HALF_PI - 2.0 * q * h, ay * h)
    phi = jnp.where(ycl < 0.0, -r2, r2)

    xi = theta * (_W / (2.0 * _PI)) + (_W - 1) * 0.5
    yi = phi * (_H / _PI) + (_H - 1) * 0.5

    x0f = jnp.floor(xi)
    y0f = jnp.floor(yi)
    fx = xi - x0f
    fy = yi - y0f
    xi0 = x0f.astype(jnp.int32)
    yi0 = y0f.astype(jnp.int32)
    xi1 = xi0 + 1
    yi1 = yi0 + 1
    vx0 = (xi0 >= 0) & (xi0 < _W)
    vx1 = (xi1 >= 0) & (xi1 < _W)
    vy0 = (yi0 >= 0) & (yi0 < _H)
    vy1 = (yi1 >= 0) & (yi1 < _H)
    xc0 = jnp.clip(xi0, 0, _W - 1)
    xc1 = jnp.clip(xi1, 0, _W - 1)
    yb0 = jnp.clip(yi0, 0, _H - 1) * _W
    yb1 = jnp.clip(yi1, 0, _H - 1) * _W
    wx0 = 1.0 - fx
    wy0 = 1.0 - fy

    f32 = lambda b: b.astype(jnp.float32)
    i00_ref[...] = yb0 + xc0
    i10_ref[...] = yb0 + xc1
    i01_ref[...] = yb1 + xc0
    i11_ref[...] = yb1 + xc1
    w00_ref[...] = wx0 * wy0 * f32(vx0 & vy0)
    w10_ref[...] = fx * wy0 * f32(vx1 & vy0)
    w01_ref[...] = wx0 * fy * f32(vx0 & vy1)
    w11_ref[...] = fx * fy * f32(vx1 & vy1)


_TB = 16384


def _tc_coords(dirT):
    grid = (_N // _TB,)
    out_spec = pl.BlockSpec((_TB,), lambda i: (i,))
    o_i = jax.ShapeDtypeStruct((_N,), jnp.int32)
    o_f = jax.ShapeDtypeStruct((_N,), jnp.float32)
    return pl.pallas_call(
        _tc_coords_body,
        grid=grid,
        in_specs=[pl.BlockSpec((3, _TB), lambda i: (0, i))],
        out_specs=[out_spec] * 8,
        out_shape=[o_i, o_i, o_i, o_i, o_f, o_f, o_f, o_f],
    )(dirT)


_NCORES = 2
_NSUB = 16
_NW = _NCORES * _NSUB
_P = _N // _NW
_CH = 1024
_MB = 128
_NMB = _CH // _MB
_NCH = _P // _CH


def _sc_gather_impl(tex_ref, i00, i10, i01, i11, w00, w10, w01, w11,
                    out_ref, idx_v, w_v, rows_v, out_v,
                    sem_g0, sem_g1, sem_o):
    wid = lax.axis_index("s") * _NCORES + lax.axis_index("c")
    base_w = wid * _P
    idx_refs = (i00, i10, i01, i11)
    w_refs = (w00, w10, w01, w11)
    sems = (sem_g0, sem_g1)
    iota16 = lax.iota(jnp.int32, 16)

    def fire(m):
        return [
            pltpu.async_copy(
                tex_ref.at[idx_v.at[c, pl.ds(m * _MB, _MB)]],
                rows_v.at[m % 2, c], sems[m % 2])
            for c in range(4)
        ]

    def chunk_body(ch, carry):
        base = base_w + ch * _CH
        par = ch & 1
        for c in range(4):
            pltpu.sync_copy(idx_refs[c].at[pl.ds(base, _CH)], idx_v.at[c])
            pltpu.sync_copy(w_refs[c].at[pl.ds(base, _CH)], w_v.at[c])

        pending = fire(0)
        for m in range(_NMB):
            nxt = fire(m + 1) if m + 1 < _NMB else []
            for cp in pending:
                cp.wait()
            pending = nxt

            def group(g, carry2, m=m):
                j0 = m * _MB + g * 16
                i0 = g * 16
                wv = [w_v[c, pl.ds(j0, 16)] for c in range(4)]
                for p in range(16):
                    acc = wv[0][p] * rows_v[m % 2, 0, i0 + p, :]
                    acc = acc + wv[1][p] * rows_v[m % 2, 1, i0 + p, :]
                    acc = acc + wv[2][p] * rows_v[m % 2, 2, i0 + p, :]
                    acc = acc + wv[3][p] * rows_v[m % 2, 3, i0 + p, :]
                    out_v[par, j0 + p, :] = acc
                return carry2

            lax.fori_loop(0, _MB // 16, group, 0)

        @pl.when(ch > 0)
        def _():
            pltpu.make_async_copy(
                out_v.at[0], out_ref.at[pl.ds(0, _CH)], sem_o).wait()

        pltpu.async_copy(out_v.at[par], out_ref.at[pl.ds(base, _CH)], sem_o)
        return carry

    lax.fori_loop(0, _NCH, chunk_body, 0)
    pltpu.make_async_copy(
        out_v.at[0], out_ref.at[pl.ds(0, _CH)], sem_o).wait()


@functools.cache
def _make_sc_gather():
    return pl.kernel(
        _sc_gather_impl,
        mesh=plsc.VectorSubcoreMesh(core_axis_name="c", subcore_axis_name="s"),
        compiler_params=pltpu.CompilerParams(use_tc_tiling_on_sc=False),
        out_type=jax.ShapeDtypeStruct((_N, _C), jnp.float32),
        scratch_types=[
            pltpu.VMEM((4, _CH), jnp.int32),
            pltpu.VMEM((4, _CH), jnp.float32),
            pltpu.VMEM((2, 4, _MB, _C), jnp.float32),
            pltpu.VMEM((2, _CH, _C), jnp.float32),
            pltpu.SemaphoreType.DMA,
            pltpu.SemaphoreType.DMA,
            pltpu.SemaphoreType.DMA,
        ],
    )


def kernel(dir, texture):
    dirT = dir.T
    i00, i10, i01, i11, w00, w10, w01, w11 = _tc_coords(dirT)
    tex2 = texture.reshape(_C, _HW).T
    return _make_sc_gather()(tex2, i00, i10, i01, i11, w00, w10, w01, w11)


# speedup vs baseline: 1.5658x; 1.5658x over previous
import functools
import math

import jax
import jax.numpy as jnp
from jax import lax
from jax.experimental import pallas as pl
from jax.experimental.pallas import tpu as pltpu
from jax.experimental.pallas import tpu_sc as plsc

_N = 2097152
_C, _H, _W = 16, 256, 256
_HW = _H * _W

_PI = math.pi
_HALF_PI = math.pi / 2.0

_ATAN7 = (
    0.9999997406246051, -0.33331371051979747, 0.1996335468228588,
    -0.13991705989210754, 0.09853834258325642, -0.05880045262807415,
    0.02386837377624885, -0.004610804113185697,
)
_ASIN5 = (
    0.999999990618061, 0.1666682132063969, 0.07493977116698855,
    0.04556902120968854, 0.023874653067893874, 0.04248026658741398,
)


def _poly(coefs, s):
    acc = jnp.full_like(s, coefs[-1])
    for c in coefs[-2::-1]:
        acc = acc * s + c
    return acc


def _tc_coords_body(d_ref, i00_ref, i10_ref, i01_ref, i11_ref,
                    w00_ref, w10_ref, w01_ref, w11_ref):
    x = d_ref[0]
    y = d_ref[1]
    z = d_ref[2]

    ax = jnp.abs(x)
    az = jnp.abs(z)
    num = jnp.minimum(ax, az)
    den = jnp.maximum(ax, az)
    den = jnp.where(den == 0.0, 1.0, den)
    t = num / den
    p = t * _poly(_ATAN7, t * t)
    r = jnp.where(ax > az, _HALF_PI - p, p)
    r = jnp.where(z < 0.0, _PI - r, r)
    theta = jnp.where(x < 0.0, -r, r)

    ycl = jnp.clip(y, -1.0, 1.0)
    ay = jnp.abs(ycl)
    big = ay > 0.5
    s2 = jnp.where(big, (1.0 - ay) * 0.5, ay * ay)
    h = _poly(_ASIN5, s2)
    q = jnp.sqrt(s2)
    r2 = jnp.where(big, _HALF_PI - 2.0 * q * h, ay * h)
    phi = jnp.where(ycl < 0.0, -r2, r2)

    xi = theta * (_W / (2.0 * _PI)) + (_W - 1) * 0.5
    yi = phi * (_H / _PI) + (_H - 1) * 0.5

    x0f = jnp.floor(xi)
    y0f = jnp.floor(yi)
    fx = xi - x0f
    fy = yi - y0f
    xi0 = x0f.astype(jnp.int32)
    yi0 = y0f.astype(jnp.int32)
    xi1 = xi0 + 1
    yi1 = yi0 + 1
    vx0 = (xi0 >= 0) & (xi0 < _W)
    vx1 = (xi1 >= 0) & (xi1 < _W)
    vy0 = (yi0 >= 0) & (yi0 < _H)
    vy1 = (yi1 >= 0) & (yi1 < _H)
    xc0 = jnp.clip(xi0, 0, _W - 1)
    xc1 = jnp.clip(xi1, 0, _W - 1)
    yb0 = jnp.clip(yi0, 0, _H - 1) * _W
    yb1 = jnp.clip(yi1, 0, _H - 1) * _W
    wx0 = 1.0 - fx
    wy0 = 1.0 - fy

    f32 = lambda b: b.astype(jnp.float32)
    i00_ref[...] = yb0 + xc0
    i10_ref[...] = yb0 + xc1
    i01_ref[...] = yb1 + xc0
    i11_ref[...] = yb1 + xc1
    w00_ref[...] = wx0 * wy0 * f32(vx0 & vy0)
    w10_ref[...] = fx * wy0 * f32(vx1 & vy0)
    w01_ref[...] = wx0 * fy * f32(vx0 & vy1)
    w11_ref[...] = fx * fy * f32(vx1 & vy1)


_TB = 16384


def _tc_coords(dirT):
    grid = (_N // _TB,)
    out_spec = pl.BlockSpec((_TB,), lambda i: (i,))
    o_i = jax.ShapeDtypeStruct((_N,), jnp.int32)
    o_f = jax.ShapeDtypeStruct((_N,), jnp.float32)
    return pl.pallas_call(
        _tc_coords_body,
        grid=grid,
        in_specs=[pl.BlockSpec((3, _TB), lambda i: (0, i))],
        out_specs=[out_spec] * 8,
        out_shape=[o_i, o_i, o_i, o_i, o_f, o_f, o_f, o_f],
    )(dirT)


_NCORES = 2
_NSUB = 16
_NW = _NCORES * _NSUB
_P = _N // _NW
_CH = 1024
_MB = 128
_NMB = _CH // _MB
_NCH = _P // _CH


def _sc_gather_impl(tex_ref, i00, i10, i01, i11, w00, w10, w01, w11,
                    out_ref, idx_v, w_v, rows_v, out_v,
                    sem_g0, sem_g1, sem_o):
    wid = lax.axis_index("s") * _NCORES + lax.axis_index("c")
    base_w = wid * _P
    idx_refs = (i00, i10, i01, i11)
    w_refs = (w00, w10, w01, w11)
    sems = (sem_g0, sem_g1)
    iota16 = lax.iota(jnp.int32, 16)

    def fire(m):
        return [
            pltpu.async_copy(
                tex_ref.at[idx_v.at[c, pl.ds(m * _MB, _MB)]],
                rows_v.at[m % 2, c], sems[m % 2])
            for c in range(4)
        ]

    def chunk_body(ch, carry):
        base = base_w + ch * _CH
        par = ch & 1
        for c in range(4):
            pltpu.sync_copy(idx_refs[c].at[pl.ds(base, _CH)], idx_v.at[c])
            pltpu.sync_copy(w_refs[c].at[pl.ds(base, _CH)], w_v.at[c])

        pending = fire(0)
        out_cps = []
        for m in range(_NMB):
            nxt = fire(m + 1) if m + 1 < _NMB else []
            for cp in pending:
                cp.wait()
            pending = nxt

            def group(g, carry2, m=m):
                j0 = m * _MB + g * 16
                i0 = g * 16
                wv = [w_v[c, pl.ds(j0, 16)] for c in range(4)]
                for p in range(16):
                    acc = wv[0][p] * rows_v[m % 2, 0, i0 + p, :]
                    acc = acc + wv[1][p] * rows_v[m % 2, 1, i0 + p, :]
                    acc = acc + wv[2][p] * rows_v[m % 2, 2, i0 + p, :]
                    acc = acc + wv[3][p] * rows_v[m % 2, 3, i0 + p, :]
                    out_v[j0 + p, :] = acc
                return carry2

            lax.fori_loop(0, _MB // 16, group, 0)
            out_cps.append(pltpu.async_copy(
                out_v.at[pl.ds(m * _MB, _MB)],
                out_ref.at[pl.ds(base + m * _MB, _MB), pl.ds(0, _C)],
                sem_o))
        for cp in out_cps:
            cp.wait()
        return carry

    lax.fori_loop(0, _NCH, chunk_body, 0)


@functools.cache
def _make_sc_gather():
    return pl.kernel(
        _sc_gather_impl,
        mesh=plsc.VectorSubcoreMesh(core_axis_name="c", subcore_axis_name="s"),
        compiler_params=pltpu.CompilerParams(use_tc_tiling_on_sc=False),
        out_type=jax.ShapeDtypeStruct((_N, 128), jnp.float32),
        scratch_types=[
            pltpu.VMEM((4, _CH), jnp.int32),
            pltpu.VMEM((4, _CH), jnp.float32),
            pltpu.VMEM((2, 4, _MB, _C), jnp.float32),
            pltpu.VMEM((_CH, _C), jnp.float32),
            pltpu.SemaphoreType.DMA,
            pltpu.SemaphoreType.DMA,
            pltpu.SemaphoreType.DMA,
        ],
    )


def kernel(dir, texture):
    dirT = dir.T
    i00, i10, i01, i11, w00, w10, w01, w11 = _tc_coords(dirT)
    tex2 = texture.reshape(_C, _HW).T
    out = _make_sc_gather()(tex2, i00, i10, i01, i11, w00, w10, w01, w11)
    return out[:, :_C]


# speedup vs baseline: 2.0963x; 1.3388x over previous
import functools
import math

import jax
import jax.numpy as jnp
from jax import lax
from jax.experimental import pallas as pl
from jax.experimental.pallas import tpu as pltpu
from jax.experimental.pallas import tpu_sc as plsc

_N = 2097152
_C, _H, _W = 16, 256, 256
_HW = _H * _W

_PI = math.pi
_HALF_PI = math.pi / 2.0

_ATAN7 = (
    0.9999997406246051, -0.33331371051979747, 0.1996335468228588,
    -0.13991705989210754, 0.09853834258325642, -0.05880045262807415,
    0.02386837377624885, -0.004610804113185697,
)
_ASIN5 = (
    0.999999990618061, 0.1666682132063969, 0.07493977116698855,
    0.04556902120968854, 0.023874653067893874, 0.04248026658741398,
)


def _poly(coefs, s):
    acc = jnp.full_like(s, coefs[-1])
    for c in coefs[-2::-1]:
        acc = acc * s + c
    return acc


def _tc_coords_body(d_ref, i00_ref, i10_ref, i01_ref, i11_ref,
                    w00_ref, w10_ref, w01_ref, w11_ref):
    x = d_ref[0]
    y = d_ref[1]
    z = d_ref[2]

    ax = jnp.abs(x)
    az = jnp.abs(z)
    num = jnp.minimum(ax, az)
    den = jnp.maximum(ax, az)
    den = jnp.where(den == 0.0, 1.0, den)
    t = num / den
    p = t * _poly(_ATAN7, t * t)
    r = jnp.where(ax > az, _HALF_PI - p, p)
    r = jnp.where(z < 0.0, _PI - r, r)
    theta = jnp.where(x < 0.0, -r, r)

    ycl = jnp.clip(y, -1.0, 1.0)
    ay = jnp.abs(ycl)
    big = ay > 0.5
    s2 = jnp.where(big, (1.0 - ay) * 0.5, ay * ay)
    h = _poly(_ASIN5, s2)
    q = jnp.sqrt(s2)
    r2 = jnp.where(big, _HALF_PI - 2.0 * q * h, ay * h)
    phi = jnp.where(ycl < 0.0, -r2, r2)

    xi = theta * (_W / (2.0 * _PI)) + (_W - 1) * 0.5
    yi = phi * (_H / _PI) + (_H - 1) * 0.5

    x0f = jnp.floor(xi)
    y0f = jnp.floor(yi)
    fx = xi - x0f
    fy = yi - y0f
    xi0 = x0f.astype(jnp.int32)
    yi0 = y0f.astype(jnp.int32)
    xi1 = xi0 + 1
    yi1 = yi0 + 1
    vx0 = (xi0 >= 0) & (xi0 < _W)
    vx1 = (xi1 >= 0) & (xi1 < _W)
    vy0 = (yi0 >= 0) & (yi0 < _H)
    vy1 = (yi1 >= 0) & (yi1 < _H)
    xc0 = jnp.clip(xi0, 0, _W - 1)
    xc1 = jnp.clip(xi1, 0, _W - 1)
    yb0 = jnp.clip(yi0, 0, _H - 1) * _W
    yb1 = jnp.clip(yi1, 0, _H - 1) * _W
    wx0 = 1.0 - fx
    wy0 = 1.0 - fy

    f32 = lambda b: b.astype(jnp.float32)
    i00_ref[...] = yb0 + xc0
    i10_ref[...] = yb0 + xc1
    i01_ref[...] = yb1 + xc0
    i11_ref[...] = yb1 + xc1
    w00_ref[...] = wx0 * wy0 * f32(vx0 & vy0)
    w10_ref[...] = fx * wy0 * f32(vx1 & vy0)
    w01_ref[...] = wx0 * fy * f32(vx0 & vy1)
    w11_ref[...] = fx * fy * f32(vx1 & vy1)


_TB = 16384


def _tc_coords(dirT):
    grid = (_N // _TB,)
    out_spec = pl.BlockSpec((_TB,), lambda i: (i,))
    o_i = jax.ShapeDtypeStruct((_N,), jnp.int32)
    o_f = jax.ShapeDtypeStruct((_N,), jnp.float32)
    return pl.pallas_call(
        _tc_coords_body,
        grid=grid,
        in_specs=[pl.BlockSpec((3, _TB), lambda i: (0, i))],
        out_specs=[out_spec] * 8,
        out_shape=[o_i, o_i, o_i, o_i, o_f, o_f, o_f, o_f],
    )(dirT)


_NCORES = 2
_NSUB = 16
_NW = _NCORES * _NSUB
_P = _N // _NW
_CH = 1024
_MB = 128
_NMB = _CH // _MB
_NCH = _P // _CH


def _sc_gather_impl(tex_ref, i00, i10, i01, i11, w00, w10, w01, w11,
                    out_ref, tex_sh, idx_v, w_v, rows_v, out_v,
                    sem_g0, sem_g1, sem_o):
    sid = lax.axis_index("s")
    wid = sid * _NCORES + lax.axis_index("c")
    base_w = wid * _P
    idx_refs = (i00, i10, i01, i11)
    w_refs = (w00, w10, w01, w11)
    sems = (sem_g0, sem_g1)

    @pl.when(sid == 0)
    def _():
        pltpu.sync_copy(tex_ref, tex_sh)
    plsc.subcore_barrier()

    def fire(m):
        return [
            pltpu.async_copy(
                tex_sh.at[idx_v.at[c, pl.ds(m * _MB, _MB)]],
                rows_v.at[m % 2, c], sems[m % 2])
            for c in range(4)
        ]

    def chunk_body(ch, carry):
        base = base_w + ch * _CH
        par = ch & 1
        for c in range(4):
            pltpu.sync_copy(idx_refs[c].at[pl.ds(base, _CH)], idx_v.at[c])
            pltpu.sync_copy(w_refs[c].at[pl.ds(base, _CH)], w_v.at[c])

        pending = fire(0)
        out_cps = []
        for m in range(_NMB):
            nxt = fire(m + 1) if m + 1 < _NMB else []
            for cp in pending:
                cp.wait()
            pending = nxt

            def group(g, carry2, m=m):
                j0 = m * _MB + g * 16
                i0 = g * 16
                wv = [w_v[c, pl.ds(j0, 16)] for c in range(4)]
                for p in range(16):
                    acc = wv[0][p] * rows_v[m % 2, 0, i0 + p, :]
                    acc = acc + wv[1][p] * rows_v[m % 2, 1, i0 + p, :]
                    acc = acc + wv[2][p] * rows_v[m % 2, 2, i0 + p, :]
                    acc = acc + wv[3][p] * rows_v[m % 2, 3, i0 + p, :]
                    out_v[j0 + p, :] = acc
                return carry2

            lax.fori_loop(0, _MB // 16, group, 0)
            out_cps.append(pltpu.async_copy(
                out_v.at[pl.ds(m * _MB, _MB)],
                out_ref.at[pl.ds(base + m * _MB, _MB), pl.ds(0, _C)],
                sem_o))
        for cp in out_cps:
            cp.wait()
        return carry

    lax.fori_loop(0, _NCH, chunk_body, 0)


@functools.cache
def _make_sc_gather():
    return pl.kernel(
        _sc_gather_impl,
        mesh=plsc.VectorSubcoreMesh(core_axis_name="c", subcore_axis_name="s"),
        compiler_params=pltpu.CompilerParams(use_tc_tiling_on_sc=False),
        out_type=jax.ShapeDtypeStruct((_N, 128), jnp.float32),
        scratch_types=[
            pltpu.VMEM_SHARED((_HW, _C), jnp.float32),
            pltpu.VMEM((4, _CH), jnp.int32),
            pltpu.VMEM((4, _CH), jnp.float32),
            pltpu.VMEM((2, 4, _MB, _C), jnp.float32),
            pltpu.VMEM((_CH, _C), jnp.float32),
            pltpu.SemaphoreType.DMA,
            pltpu.SemaphoreType.DMA,
            pltpu.SemaphoreType.DMA,
        ],
    )


def kernel(dir, texture):
    dirT = dir.T
    i00, i10, i01, i11, w00, w10, w01, w11 = _tc_coords(dirT)
    tex2 = texture.reshape(_C, _HW).T
    out = _make_sc_gather()(tex2, i00, i10, i01, i11, w00, w10, w01, w11)
    return out[:, :_C]


# speedup vs baseline: 2.1059x; 1.0046x over previous
import functools
import math

import jax
import jax.numpy as jnp
from jax import lax
from jax.experimental import pallas as pl
from jax.experimental.pallas import tpu as pltpu
from jax.experimental.pallas import tpu_sc as plsc

_N = 2097152
_C, _H, _W = 16, 256, 256
_HW = _H * _W

_PI = math.pi
_HALF_PI = math.pi / 2.0

_ATAN7 = (
    0.9999997406246051, -0.33331371051979747, 0.1996335468228588,
    -0.13991705989210754, 0.09853834258325642, -0.05880045262807415,
    0.02386837377624885, -0.004610804113185697,
)
_ASIN5 = (
    0.999999990618061, 0.1666682132063969, 0.07493977116698855,
    0.04556902120968854, 0.023874653067893874, 0.04248026658741398,
)


def _poly(coefs, s):
    acc = jnp.full_like(s, coefs[-1])
    for c in coefs[-2::-1]:
        acc = acc * s + c
    return acc


def _tc_coords_body(d_ref, i00_ref, i10_ref, i01_ref, i11_ref,
                    w00_ref, w10_ref, w01_ref, w11_ref):
    x = d_ref[0]
    y = d_ref[1]
    z = d_ref[2]

    ax = jnp.abs(x)
    az = jnp.abs(z)
    num = jnp.minimum(ax, az)
    den = jnp.maximum(ax, az)
    den = jnp.where(den == 0.0, 1.0, den)
    t = num / den
    p = t * _poly(_ATAN7, t * t)
    r = jnp.where(ax > az, _HALF_PI - p, p)
    r = jnp.where(z < 0.0, _PI - r, r)
    theta = jnp.where(x < 0.0, -r, r)

    ycl = jnp.clip(y, -1.0, 1.0)
    ay = jnp.abs(ycl)
    big = ay > 0.5
    s2 = jnp.where(big, (1.0 - ay) * 0.5, ay * ay)
    h = _poly(_ASIN5, s2)
    q = jnp.sqrt(s2)
    r2 = jnp.where(big, _HALF_PI - 2.0 * q * h, ay * h)
    phi = jnp.where(ycl < 0.0, -r2, r2)

    xi = theta * (_W / (2.0 * _PI)) + (_W - 1) * 0.5
    yi = phi * (_H / _PI) + (_H - 1) * 0.5

    x0f = jnp.floor(xi)
    y0f = jnp.floor(yi)
    fx = xi - x0f
    fy = yi - y0f
    xi0 = x0f.astype(jnp.int32)
    yi0 = y0f.astype(jnp.int32)
    xi1 = xi0 + 1
    yi1 = yi0 + 1
    vx0 = (xi0 >= 0) & (xi0 < _W)
    vx1 = (xi1 >= 0) & (xi1 < _W)
    vy0 = (yi0 >= 0) & (yi0 < _H)
    vy1 = (yi1 >= 0) & (yi1 < _H)
    xc0 = jnp.clip(xi0, 0, _W - 1)
    xc1 = jnp.clip(xi1, 0, _W - 1)
    yb0 = jnp.clip(yi0, 0, _H - 1) * _W
    yb1 = jnp.clip(yi1, 0, _H - 1) * _W
    wx0 = 1.0 - fx
    wy0 = 1.0 - fy

    f32 = lambda b: b.astype(jnp.float32)
    i00_ref[...] = yb0 + xc0
    i10_ref[...] = yb0 + xc1
    i01_ref[...] = yb1 + xc0
    i11_ref[...] = yb1 + xc1
    w00_ref[...] = wx0 * wy0 * f32(vx0 & vy0)
    w10_ref[...] = fx * wy0 * f32(vx1 & vy0)
    w01_ref[...] = wx0 * fy * f32(vx0 & vy1)
    w11_ref[...] = fx * fy * f32(vx1 & vy1)


_TB = 16384


def _tc_coords(dirT):
    grid = (_N // _TB,)
    out_spec = pl.BlockSpec((_TB,), lambda i: (i,))
    o_i = jax.ShapeDtypeStruct((_N,), jnp.int32)
    o_f = jax.ShapeDtypeStruct((_N,), jnp.float32)
    return pl.pallas_call(
        _tc_coords_body,
        grid=grid,
        in_specs=[pl.BlockSpec((3, _TB), lambda i: (0, i))],
        out_specs=[out_spec] * 8,
        out_shape=[o_i, o_i, o_i, o_i, o_f, o_f, o_f, o_f],
    )(dirT)


_NCORES = 2
_NSUB = 16
_NW = _NCORES * _NSUB
_P = _N // _NW
_CH = 1024
_MB = 128
_NMB = _CH // _MB
_NCH = _P // _CH
_DEPTH = 3
_NBUF = 4


def _sc_gather_impl(tex_ref, i00, i10, i01, i11, w00, w10, w01, w11,
                    out_ref, tex_sh, idx_v, w_v, rows_v, out_v,
                    sem_g0, sem_g1, sem_g2, sem_g3, sem_o):
    sid = lax.axis_index("s")
    wid = sid * _NCORES + lax.axis_index("c")
    base_w = wid * _P
    idx_refs = (i00, i10, i01, i11)
    w_refs = (w00, w10, w01, w11)
    sems = (sem_g0, sem_g1, sem_g2, sem_g3)

    @pl.when(sid == 0)
    def _():
        pltpu.sync_copy(tex_ref, tex_sh)
    plsc.subcore_barrier()

    def fire(m):
        return [
            pltpu.async_copy(
                tex_sh.at[idx_v.at[c, pl.ds(m * _MB, _MB)]],
                rows_v.at[m % _NBUF, c], sems[m % _NBUF])
            for c in range(4)
        ]

    def chunk_body(ch, carry):
        base = base_w + ch * _CH
        for c in range(4):
            pltpu.sync_copy(idx_refs[c].at[pl.ds(base, _CH)], idx_v.at[c])
            pltpu.sync_copy(w_refs[c].at[pl.ds(base, _CH)], w_v.at[c])

        pending = [fire(m) for m in range(_DEPTH)]
        out_cps = []
        for m in range(_NMB):
            for cp in pending[0]:
                cp.wait()
            pending = pending[1:]
            if m + _DEPTH < _NMB:
                pending.append(fire(m + _DEPTH))

            def group(g, carry2, m=m):
                j0 = m * _MB + g * 16
                i0 = g * 16
                wv = [w_v[c, pl.ds(j0, 16)] for c in range(4)]
                for p in range(16):
                    acc = wv[0][p] * rows_v[m % _NBUF, 0, i0 + p, :]
                    acc = acc + wv[1][p] * rows_v[m % _NBUF, 1, i0 + p, :]
                    acc = acc + wv[2][p] * rows_v[m % _NBUF, 2, i0 + p, :]
                    acc = acc + wv[3][p] * rows_v[m % _NBUF, 3, i0 + p, :]
                    out_v[j0 + p, :] = acc
                return carry2

            lax.fori_loop(0, _MB // 16, group, 0)
            out_cps.append(pltpu.async_copy(
                out_v.at[pl.ds(m * _MB, _MB)],
                out_ref.at[pl.ds(base + m * _MB, _MB), pl.ds(0, _C)],
                sem_o))
        for cp in out_cps:
            cp.wait()
        return carry

    lax.fori_loop(0, _NCH, chunk_body, 0)


@functools.cache
def _make_sc_gather():
    return pl.kernel(
        _sc_gather_impl,
        mesh=plsc.VectorSubcoreMesh(core_axis_name="c", subcore_axis_name="s"),
        compiler_params=pltpu.CompilerParams(use_tc_tiling_on_sc=False),
        out_type=jax.ShapeDtypeStruct((_N, 128), jnp.float32),
        scratch_types=[
            pltpu.VMEM_SHARED((_HW, _C), jnp.float32),
            pltpu.VMEM((4, _CH), jnp.int32),
            pltpu.VMEM((4, _CH), jnp.float32),
            pltpu.VMEM((_NBUF, 4, _MB, _C), jnp.float32),
            pltpu.VMEM((_CH, _C), jnp.float32),
            pltpu.SemaphoreType.DMA,
            pltpu.SemaphoreType.DMA,
            pltpu.SemaphoreType.DMA,
            pltpu.SemaphoreType.DMA,
            pltpu.SemaphoreType.DMA,
        ],
    )


def kernel(dir, texture):
    dirT = dir.T
    i00, i10, i01, i11, w00, w10, w01, w11 = _tc_coords(dirT)
    tex2 = texture.reshape(_C, _HW).T
    out = _make_sc_gather()(tex2, i00, i10, i01, i11, w00, w10, w01, w11)
    return out[:, :_C]


# speedup vs baseline: 2.1097x; 1.0018x over previous
import functools
import math

import jax
import jax.numpy as jnp
from jax import lax
from jax.experimental import pallas as pl
from jax.experimental.pallas import tpu as pltpu
from jax.experimental.pallas import tpu_sc as plsc

_N = 2097152
_C, _H, _W = 16, 256, 256
_HW = _H * _W

_PI = math.pi
_HALF_PI = math.pi / 2.0

_ATAN7 = (
    0.9999997406246051, -0.33331371051979747, 0.1996335468228588,
    -0.13991705989210754, 0.09853834258325642, -0.05880045262807415,
    0.02386837377624885, -0.004610804113185697,
)
_ASIN5 = (
    0.999999990618061, 0.1666682132063969, 0.07493977116698855,
    0.04556902120968854, 0.023874653067893874, 0.04248026658741398,
)


_SPLAT_DN = lax.GatherDimensionNumbers(
    offset_dims=(), collapsed_slice_dims=(0,), start_index_map=(0,))


def _lane_splat(v, lane):
    return lax.gather(v, lane[:, None], _SPLAT_DN, slice_sizes=(1,),
                      mode=lax.GatherScatterMode.PROMISE_IN_BOUNDS)


def _poly(coefs, s):
    acc = jnp.full_like(s, coefs[-1])
    for c in coefs[-2::-1]:
        acc = acc * s + c
    return acc


def _tc_coords_body(d_ref, i00_ref, i10_ref, i01_ref, i11_ref,
                    w00_ref, w10_ref, w01_ref, w11_ref):
    x = d_ref[0]
    y = d_ref[1]
    z = d_ref[2]

    ax = jnp.abs(x)
    az = jnp.abs(z)
    num = jnp.minimum(ax, az)
    den = jnp.maximum(ax, az)
    den = jnp.where(den == 0.0, 1.0, den)
    t = num / den
    p = t * _poly(_ATAN7, t * t)
    r = jnp.where(ax > az, _HALF_PI - p, p)
    r = jnp.where(z < 0.0, _PI - r, r)
    theta = jnp.where(x < 0.0, -r, r)

    ycl = jnp.clip(y, -1.0, 1.0)
    ay = jnp.abs(ycl)
    big = ay > 0.5
    s2 = jnp.where(big, (1.0 - ay) * 0.5, ay * ay)
    h = _poly(_ASIN5, s2)
    q = jnp.sqrt(s2)
    r2 = jnp.where(big, _HALF_PI - 2.0 * q * h, ay * h)
    phi = jnp.where(ycl < 0.0, -r2, r2)

    xi = theta * (_W / (2.0 * _PI)) + (_W - 1) * 0.5
    yi = phi * (_H / _PI) + (_H - 1) * 0.5

    x0f = jnp.floor(xi)
    y0f = jnp.floor(yi)
    fx = xi - x0f
    fy = yi - y0f
    xi0 = x0f.astype(jnp.int32)
    yi0 = y0f.astype(jnp.int32)
    xi1 = xi0 + 1
    yi1 = yi0 + 1
    vx0 = (xi0 >= 0) & (xi0 < _W)
    vx1 = (xi1 >= 0) & (xi1 < _W)
    vy0 = (yi0 >= 0) & (yi0 < _H)
    vy1 = (yi1 >= 0) & (yi1 < _H)
    xc0 = jnp.clip(xi0, 0, _W - 1)
    xc1 = jnp.clip(xi1, 0, _W - 1)
    yb0 = jnp.clip(yi0, 0, _H - 1) * _W
    yb1 = jnp.clip(yi1, 0, _H - 1) * _W
    wx0 = 1.0 - fx
    wy0 = 1.0 - fy

    f32 = lambda b: b.astype(jnp.float32)
    i00_ref[...] = yb0 + xc0
    i10_ref[...] = yb0 + xc1
    i01_ref[...] = yb1 + xc0
    i11_ref[...] = yb1 + xc1
    w00_ref[...] = wx0 * wy0 * f32(vx0 & vy0)
    w10_ref[...] = fx * wy0 * f32(vx1 & vy0)
    w01_ref[...] = wx0 * fy * f32(vx0 & vy1)
    w11_ref[...] = fx * fy * f32(vx1 & vy1)


_TB = 16384


def _tc_coords(dirT):
    grid = (_N // _TB,)
    out_spec = pl.BlockSpec((_TB,), lambda i: (i,))
    o_i = jax.ShapeDtypeStruct((_N,), jnp.int32)
    o_f = jax.ShapeDtypeStruct((_N,), jnp.float32)
    return pl.pallas_call(
        _tc_coords_body,
        grid=grid,
        in_specs=[pl.BlockSpec((3, _TB), lambda i: (0, i))],
        out_specs=[out_spec] * 8,
        out_shape=[o_i, o_i, o_i, o_i, o_f, o_f, o_f, o_f],
    )(dirT)


_NCORES = 2
_NSUB = 16
_NW = _NCORES * _NSUB
_P = _N // _NW
_CH = 1024
_MB = 128
_NMB = _CH // _MB
_NCH = _P // _CH
_DEPTH = 3
_NBUF = 4


def _sc_gather_impl(tex_ref, i00, i10, i01, i11, w00, w10, w01, w11,
                    out_ref, tex_sh, idx_v, w_v, rows_v, out_v,
                    sem_g0, sem_g1, sem_g2, sem_g3, sem_o):
    sid = lax.axis_index("s")
    wid = sid * _NCORES + lax.axis_index("c")
    base_w = wid * _P
    idx_refs = (i00, i10, i01, i11)
    w_refs = (w00, w10, w01, w11)
    sems = (sem_g0, sem_g1, sem_g2, sem_g3)

    @pl.when(sid == 0)
    def _():
        pltpu.sync_copy(tex_ref, tex_sh)
    plsc.subcore_barrier()

    def fire(m):
        return [
            pltpu.async_copy(
                tex_sh.at[idx_v.at[c, pl.ds(m * _MB, _MB)]],
                rows_v.at[m % _NBUF, c], sems[m % _NBUF])
            for c in range(4)
        ]

    def chunk_body(ch, carry):
        base = base_w + ch * _CH
        for c in range(4):
            pltpu.sync_copy(idx_refs[c].at[pl.ds(base, _CH)], idx_v.at[c])
            pltpu.sync_copy(w_refs[c].at[pl.ds(base, _CH)], w_v.at[c])

        pending = [fire(m) for m in range(_DEPTH)]
        out_cps = []
        for m in range(_NMB):
            for cp in pending[0]:
                cp.wait()
            pending = pending[1:]
            if m + _DEPTH < _NMB:
                pending.append(fire(m + _DEPTH))

            def group(g, carry2, m=m):
                j0 = m * _MB + g * 16
                i0 = g * 16
                wv = [w_v[c, pl.ds(j0, 16)] for c in range(4)]
                for p in range(16):
                    lane = jnp.full((16,), p, jnp.int32)
                    acc = (_lane_splat(wv[0], lane)
                           * rows_v[m % _NBUF, 0, i0 + p, :])
                    acc = acc + (_lane_splat(wv[1], lane)
                                 * rows_v[m % _NBUF, 1, i0 + p, :])
                    acc = acc + (_lane_splat(wv[2], lane)
                                 * rows_v[m % _NBUF, 2, i0 + p, :])
                    acc = acc + (_lane_splat(wv[3], lane)
                                 * rows_v[m % _NBUF, 3, i0 + p, :])
                    out_v[j0 + p, :] = acc
                return carry2

            lax.fori_loop(0, _MB // 16, group, 0)
            out_cps.append(pltpu.async_copy(
                out_v.at[pl.ds(m * _MB, _MB)],
                out_ref.at[pl.ds(base + m * _MB, _MB), pl.ds(0, _C)],
                sem_o))
        for cp in out_cps:
            cp.wait()
        return carry

    lax.fori_loop(0, _NCH, chunk_body, 0)


@functools.cache
def _make_sc_gather():
    return pl.kernel(
        _sc_gather_impl,
        mesh=plsc.VectorSubcoreMesh(core_axis_name="c", subcore_axis_name="s"),
        compiler_params=pltpu.CompilerParams(use_tc_tiling_on_sc=False),
        out_type=jax.ShapeDtypeStruct((_N, 128), jnp.float32),
        scratch_types=[
            pltpu.VMEM_SHARED((_HW, _C), jnp.float32),
            pltpu.VMEM((4, _CH), jnp.int32),
            pltpu.VMEM((4, _CH), jnp.float32),
            pltpu.VMEM((_NBUF, 4, _MB, _C), jnp.float32),
            pltpu.VMEM((_CH, _C), jnp.float32),
            pltpu.SemaphoreType.DMA,
            pltpu.SemaphoreType.DMA,
            pltpu.SemaphoreType.DMA,
            pltpu.SemaphoreType.DMA,
            pltpu.SemaphoreType.DMA,
        ],
    )


def kernel(dir, texture):
    dirT = dir.T
    i00, i10, i01, i11, w00, w10, w01, w11 = _tc_coords(dirT)
    tex2 = texture.reshape(_C, _HW).T
    out = _make_sc_gather()(tex2, i00, i10, i01, i11, w00, w10, w01, w11)
    return out[:, :_C]
